# Initial kernel scaffold; baseline (speedup 1.0000x reference)
import jax
import jax.numpy as jnp
from jax.experimental import pallas as pl


def kernel(seq_len, table):
    raise NotImplementedError("write your pallas kernel here")


# speedup vs baseline: 42.2044x; 42.2044x over previous
import functools

import jax
import jax.numpy as jnp
from jax import lax
from jax.experimental import pallas as pl
from jax.experimental.pallas import tpu as pltpu
from jax.experimental.pallas import tpu_sc as plsc

N_HEADS = 16
MAX_REL = 32
S = 2048
NUM_REL = 2 * MAX_REL + 1
M_LEN = 4096
N_SHIFTS = 8
RING = 8
HALF = S // 2


def _body(table_hbm, out_hbm, table_v, m_v, sem):
    info = plsc.get_sparse_core_info()
    nc = info.num_cores
    wid = lax.axis_index("s") * nc + lax.axis_index("c")
    head = wid // 2
    half = wid % 2
    i0 = half * HALF

    pltpu.sync_copy(table_hbm, table_v)

    hvec = jnp.full((16,), head, dtype=jnp.int32)
    lane = lax.iota(jnp.int32, 16)
    gdims = lax.GatherDimensionNumbers(
        offset_dims=(), collapsed_slice_dims=(0,), start_index_map=(0,)
    )

    def tval(c):
        return lax.gather(
            table_v[c],
            hvec[:, None],
            gdims,
            slice_sizes=(1,),
            mode=lax.GatherScatterMode.PROMISE_IN_BOUNDS,
        )

    t_lo = tval(0)
    t_hi = tval(NUM_REL - 1)

    for sh in range(N_SHIFTS):
        mbase = sh * M_LEN
        lo_end = 0
        hi_start = 0
        for kc in range(M_LEN // 16):
            if 16 * kc + 15 + sh - (S - 1) <= -MAX_REL:
                lo_end = kc + 1
            if 16 * kc + sh - (S - 1) < MAX_REL:
                hi_start = kc + 1

        def fill_lo(kc, carry, mbase=mbase, vec=t_lo):
            m_v[pl.ds(pl.multiple_of(mbase + kc * 16, 8), 16)] = vec
            return carry

        def fill_hi(kc, carry, mbase=mbase, vec=t_hi):
            m_v[pl.ds(pl.multiple_of(mbase + kc * 16, 8), 16)] = vec
            return carry

        lax.fori_loop(0, lo_end, fill_lo, 0)
        lax.fori_loop(hi_start, M_LEN // 16, fill_hi, 0)

        for kc in range(lo_end, hi_start):
            k0 = 16 * kc
            cvec = jnp.clip(k0 + lane + sh - (S - 1), -MAX_REL, MAX_REL) + MAX_REL
            c_min = max(0, min(NUM_REL - 1, k0 + sh - (S - 1) + MAX_REL))
            c_max = max(0, min(NUM_REL - 1, k0 + 15 + sh - (S - 1) + MAX_REL))
            val = tval(c_min)
            for c in range(c_min + 1, c_max + 1):
                val = jnp.where(cvec >= c, tval(c), val)
            m_v[pl.ds(mbase + k0, 16)] = val

    def row_copy(i_loc):
        i = i0 + i_loc
        o = (S - 1) - i
        sh = lax.rem(o, N_SHIFTS)
        base = o - sh
        return pltpu.make_async_copy(
            m_v.at[pl.ds(pl.multiple_of(sh * M_LEN + base, 8), S)],
            out_hbm.at[0, head, i, :],
            sem,
        )

    def prologue(i_loc, carry):
        row_copy(i_loc).start()
        return carry

    lax.fori_loop(0, RING, prologue, 0)

    def steady(i_loc, carry):
        row_copy(i_loc - RING).wait()
        row_copy(i_loc).start()
        return carry

    lax.fori_loop(RING, HALF, steady, 0)

    def epilogue(i_loc, carry):
        row_copy(i_loc).wait()
        return carry

    lax.fori_loop(HALF - RING, HALF, epilogue, 0)


@jax.jit
def _bias(table):
    run = pl.kernel(
        _body,
        out_type=jax.ShapeDtypeStruct((1, N_HEADS, S, S), jnp.float32),
        mesh=plsc.VectorSubcoreMesh(core_axis_name="c", subcore_axis_name="s"),
        compiler_params=pltpu.CompilerParams(use_tc_tiling_on_sc=False),
        scratch_types=[
            pltpu.VMEM((NUM_REL, 16), jnp.float32),
            pltpu.VMEM((N_SHIFTS * M_LEN,), jnp.float32),
            pltpu.SemaphoreType.DMA,
        ],
    )
    return run(table)


def kernel(seq_len, table):
    del seq_len
    return _bias(table)


# speedup vs baseline: 42.2959x; 1.0022x over previous
import functools

import jax
import jax.numpy as jnp
from jax import lax
from jax.experimental import pallas as pl
from jax.experimental.pallas import tpu as pltpu
from jax.experimental.pallas import tpu_sc as plsc

N_HEADS = 16
MAX_REL = 32
S = 2048
NUM_REL = 2 * MAX_REL + 1
M_LEN = 4096
N_SHIFTS = 8
RING = 8
HALF = S // 2


def _body(table_hbm, out_hbm, table_v, m_v, sem):
    info = plsc.get_sparse_core_info()
    nc = info.num_cores
    wid = lax.axis_index("s") * nc + lax.axis_index("c")
    head = wid // 2
    half = wid % 2
    i0 = half * HALF

    pltpu.sync_copy(table_hbm, table_v)

    hvec = jnp.full((16,), head, dtype=jnp.int32)
    lane = lax.iota(jnp.int32, 16)
    gdims = lax.GatherDimensionNumbers(
        offset_dims=(), collapsed_slice_dims=(0,), start_index_map=(0,)
    )

    def tval(c):
        return lax.gather(
            table_v[c],
            hvec[:, None],
            gdims,
            slice_sizes=(1,),
            mode=lax.GatherScatterMode.PROMISE_IN_BOUNDS,
        )

    t_lo = tval(0)
    t_hi = tval(NUM_REL - 1)

    for row in range(N_SHIFTS):
        sh = (N_SHIFTS - 1) - row
        lo_end = 0
        hi_start = 0
        for kc in range(M_LEN // 16):
            if 16 * kc + 15 + sh - (S - 1) <= -MAX_REL:
                lo_end = kc + 1
            if 16 * kc + sh - (S - 1) < MAX_REL:
                hi_start = kc + 1

        row_ref = m_v.at[row]

        def fill_lo(kc, carry, row_ref=row_ref, vec=t_lo):
            row_ref[pl.ds(pl.multiple_of(kc * 16, 8), 16)] = vec
            return carry

        def fill_hi(kc, carry, row_ref=row_ref, vec=t_hi):
            row_ref[pl.ds(pl.multiple_of(kc * 16, 8), 16)] = vec
            return carry

        lax.fori_loop(0, lo_end, fill_lo, 0)
        lax.fori_loop(hi_start, M_LEN // 16, fill_hi, 0)

        for kc in range(lo_end, hi_start):
            k0 = 16 * kc
            cvec = jnp.clip(k0 + lane + sh - (S - 1), -MAX_REL, MAX_REL) + MAX_REL
            c_min = max(0, min(NUM_REL - 1, k0 + sh - (S - 1) + MAX_REL))
            c_max = max(0, min(NUM_REL - 1, k0 + 15 + sh - (S - 1) + MAX_REL))
            val = tval(c_min)
            for c in range(c_min + 1, c_max + 1):
                val = jnp.where(cvec >= c, tval(c), val)
            row_ref[pl.ds(k0, 16)] = val

    n_blocks = HALF // N_SHIFTS

    def block_copy(b):
        i_blk = i0 + b * N_SHIFTS
        base = (S - 1) - (N_SHIFTS - 1) - i_blk
        return pltpu.make_async_copy(
            m_v.at[:, pl.ds(pl.multiple_of(base, 8), S)],
            out_hbm.at[0, head, pl.ds(pl.multiple_of(i_blk, 8), N_SHIFTS), :],
            sem,
        )

    def prologue(b, carry):
        block_copy(b).start()
        return carry

    lax.fori_loop(0, RING, prologue, 0)

    def steady(b, carry):
        block_copy(b - RING).wait()
        block_copy(b).start()
        return carry

    lax.fori_loop(RING, n_blocks, steady, 0)

    def epilogue(b, carry):
        block_copy(b).wait()
        return carry

    lax.fori_loop(n_blocks - RING, n_blocks, epilogue, 0)


@jax.jit
def _bias(table):
    run = pl.kernel(
        _body,
        out_type=jax.ShapeDtypeStruct((1, N_HEADS, S, S), jnp.float32),
        mesh=plsc.VectorSubcoreMesh(core_axis_name="c", subcore_axis_name="s"),
        compiler_params=pltpu.CompilerParams(use_tc_tiling_on_sc=False),
        scratch_types=[
            pltpu.VMEM((NUM_REL, 16), jnp.float32),
            pltpu.VMEM((N_SHIFTS, M_LEN), jnp.float32),
            pltpu.SemaphoreType.DMA,
        ],
    )
    return run(table)


def kernel(seq_len, table):
    del seq_len
    return _bias(table)


# speedup vs baseline: 142.1077x; 3.3598x over previous
import functools

import jax
import jax.numpy as jnp
from jax import lax
from jax.experimental import pallas as pl
from jax.experimental.pallas import tpu as pltpu
from jax.experimental.pallas import tpu_sc as plsc

N_HEADS = 16
MAX_REL = 32
S = 2048
NUM_REL = 2 * MAX_REL + 1
M_LEN = 4096
N_SHIFTS = 8
RING = 8
HALF = S // 2


def _body(table_hbm, out_hbm, table_v, m_v, sem):
    info = plsc.get_sparse_core_info()
    nc = info.num_cores
    wid = lax.axis_index("s") * nc + lax.axis_index("c")
    head = wid // 2
    half = wid % 2
    i0 = half * HALF

    pltpu.sync_copy(table_hbm, table_v)

    hvec = jnp.full((16,), head, dtype=jnp.int32)
    lane = lax.iota(jnp.int32, 16)
    gdims = lax.GatherDimensionNumbers(
        offset_dims=(), collapsed_slice_dims=(0,), start_index_map=(0,)
    )

    def tval(c):
        return lax.gather(
            table_v[c],
            hvec[:, None],
            gdims,
            slice_sizes=(1,),
            mode=lax.GatherScatterMode.PROMISE_IN_BOUNDS,
        )

    t_lo = tval(0)
    t_hi = tval(NUM_REL - 1)

    for row in range(N_SHIFTS):
        sh = (N_SHIFTS - 1) - row
        lo_end = 0
        hi_start = 0
        for kc in range(M_LEN // 16):
            if 16 * kc + 15 + sh - (S - 1) <= -MAX_REL:
                lo_end = kc + 1
            if 16 * kc + sh - (S - 1) < MAX_REL:
                hi_start = kc + 1

        row_ref = m_v.at[row]

        def fill_lo(kc, carry, row_ref=row_ref, vec=t_lo):
            row_ref[pl.ds(pl.multiple_of(kc * 16, 8), 16)] = vec
            return carry

        def fill_hi(kc, carry, row_ref=row_ref, vec=t_hi):
            row_ref[pl.ds(pl.multiple_of(kc * 16, 8), 16)] = vec
            return carry

        lax.fori_loop(0, lo_end, fill_lo, 0)
        lax.fori_loop(hi_start, M_LEN // 16, fill_hi, 0)

        for kc in range(lo_end, hi_start):
            k0 = 16 * kc
            cvec = jnp.clip(k0 + lane + sh - (S - 1), -MAX_REL, MAX_REL) + MAX_REL
            c_min = max(0, min(NUM_REL - 1, k0 + sh - (S - 1) + MAX_REL))
            c_max = max(0, min(NUM_REL - 1, k0 + 15 + sh - (S - 1) + MAX_REL))
            val = tval(c_min)
            for c in range(c_min + 1, c_max + 1):
                val = jnp.where(cvec >= c, tval(c), val)
            row_ref[pl.ds(k0, 16)] = val

    n_blocks = (HALF // N_SHIFTS) * (S // 128)
    i_tile0 = (half * HALF) // N_SHIFTS

    def block_copy(b):
        it = i_tile0 + (b >> 4)
        jt = b & 15
        base = (S - 8) - 8 * it + 128 * jt
        return pltpu.make_async_copy(
            m_v.at[:, pl.ds(pl.multiple_of(base, 8), 128)],
            out_hbm.at[head, it, jt],
            sem,
        )

    def prologue(b, carry):
        block_copy(b).start()
        return carry

    lax.fori_loop(0, RING, prologue, 0)

    def steady(b, carry):
        block_copy(b - RING).wait()
        block_copy(b).start()
        return carry

    lax.fori_loop(RING, n_blocks, steady, 0)

    def epilogue(b, carry):
        block_copy(b).wait()
        return carry

    lax.fori_loop(n_blocks - RING, n_blocks, epilogue, 0)


@jax.jit
def _bias(table):
    run = pl.kernel(
        _body,
        out_type=jax.ShapeDtypeStruct(
            (N_HEADS, S // 8, S // 128, 8, 128), jnp.float32
        ),
        mesh=plsc.VectorSubcoreMesh(core_axis_name="c", subcore_axis_name="s"),
        compiler_params=pltpu.CompilerParams(use_tc_tiling_on_sc=False),
        scratch_types=[
            pltpu.VMEM((NUM_REL, 16), jnp.float32),
            pltpu.VMEM((N_SHIFTS, M_LEN), jnp.float32),
            pltpu.SemaphoreType.DMA,
        ],
    )
    tiles = run(table)
    return jnp.transpose(tiles, (0, 1, 3, 2, 4)).reshape(1, N_HEADS, S, S)


def kernel(seq_len, table):
    del seq_len
    return _bias(table)
